# baseline (speedup 1.0000x reference)
import functools

import jax
import jax.numpy as jnp
from jax import lax
from jax.experimental import pallas as pl
from jax.experimental.pallas import tpu as pltpu
from jax.experimental.pallas import tpu_sc as plsc

NUM_WORKERS = 32
SHORT = 32
DIM = 128


WORDS = SHORT // 2


def _sc_gather(ids_flat, table_i32, n_rows):
    b_per_w = n_rows // NUM_WORKERS
    mesh = plsc.VectorSubcoreMesh(core_axis_name="c", subcore_axis_name="s")

    @functools.partial(
        pl.kernel,
        mesh=mesh,
        out_type=jax.ShapeDtypeStruct((n_rows, WORDS), jnp.int32),
        scratch_types=[
            pltpu.VMEM((b_per_w,), jnp.int32),
            pltpu.VMEM((b_per_w, WORDS), jnp.int32),
            pltpu.SemaphoreType.DMA,
        ],
        compiler_params=pltpu.CompilerParams(use_tc_tiling_on_sc=False),
    )
    def gather_kernel(ids_hbm, table_hbm, out_hbm, idx_v, rows_v, sem):
        wid = lax.axis_index("s") * 2 + lax.axis_index("c")
        base = wid * b_per_w
        pltpu.sync_copy(ids_hbm.at[pl.ds(base, b_per_w)], idx_v)
        pltpu.async_copy(table_hbm.at[idx_v], rows_v, sem).wait()
        pltpu.sync_copy(rows_v, out_hbm.at[pl.ds(base, b_per_w)])

    return gather_kernel(ids_flat, table_i32)


def _proj_body(x_ref, w_ref, b_ref, o_ref):
    acc = jnp.dot(x_ref[...], w_ref[...], preferred_element_type=jnp.float32)
    o_ref[...] = (acc + b_ref[...]).astype(jnp.bfloat16)


def _tc_project(x, wt, b2, n_rows):
    block = 8192
    return pl.pallas_call(
        _proj_body,
        grid=(n_rows // block,),
        in_specs=[
            pl.BlockSpec((block, SHORT), lambda i: (i, 0)),
            pl.BlockSpec((SHORT, DIM), lambda i: (0, 0)),
            pl.BlockSpec((1, DIM), lambda i: (0, 0)),
        ],
        out_specs=pl.BlockSpec((block, DIM), lambda i: (i, 0)),
        out_shape=jax.ShapeDtypeStruct((n_rows, DIM), jnp.bfloat16),
    )(x, wt, b2)


def kernel(ids, embed, W, b):
    B, L = ids.shape
    n_rows = B * L
    num_emb = embed.shape[0]
    ids_flat = ids.reshape(n_rows).astype(jnp.int32)
    e_u16 = jax.lax.bitcast_convert_type(embed, jnp.uint16).astype(jnp.uint32)
    table_i32 = (e_u16[:, 0::2] | (e_u16[:, 1::2] << 16)).astype(jnp.int32)
    x_i32 = _sc_gather(ids_flat, table_i32, n_rows)
    x = jax.lax.bitcast_convert_type(x_i32, jnp.bfloat16).reshape(n_rows, SHORT)
    wt = W.astype(jnp.bfloat16).T
    b2 = b.astype(jnp.bfloat16).reshape(1, DIM)
    out = _tc_project(x, wt, b2, n_rows)
    return out.reshape(B, L, DIM)


# speedup vs baseline: 5.4466x; 5.4466x over previous
import functools

import jax
import jax.numpy as jnp
from jax import lax
from jax.experimental import pallas as pl
from jax.experimental.pallas import tpu as pltpu
from jax.experimental.pallas import tpu_sc as plsc

NUM_WORKERS = 32
SHORT = 32
DIM = 128


WORDS = SHORT // 2


def _sc_gather(ids_flat, table, n_rows):
    b_per_w = n_rows // NUM_WORKERS
    mesh = plsc.VectorSubcoreMesh(core_axis_name="c", subcore_axis_name="s")

    @functools.partial(
        pl.kernel,
        mesh=mesh,
        out_type=jax.ShapeDtypeStruct((n_rows, WORDS), jnp.int32),
        scratch_types=[
            pltpu.VMEM((b_per_w,), jnp.int32),
            pltpu.VMEM((b_per_w, WORDS), jnp.int32),
            pltpu.SemaphoreType.DMA,
        ],
        compiler_params=pltpu.CompilerParams(use_tc_tiling_on_sc=False),
    )
    def gather_kernel(ids_hbm, table_hbm, out_hbm, idx_v, rows_v, sem):
        wid = lax.axis_index("s") * 2 + lax.axis_index("c")
        base = wid * b_per_w
        pltpu.sync_copy(ids_hbm.at[pl.ds(base, b_per_w)], idx_v)
        pltpu.async_copy(table_hbm.at[idx_v], rows_v, sem).wait()
        pltpu.sync_copy(rows_v, out_hbm.at[pl.ds(base, b_per_w)])

    return gather_kernel(ids_flat, table)


def _proj_body(x_ref, w_ref, b_ref, o_ref):
    acc = jnp.dot(x_ref[...], w_ref[...], preferred_element_type=jnp.float32)
    o_ref[...] = (acc + b_ref[...]).astype(jnp.bfloat16)


def _tc_project(x, wt, b2, n_rows):
    block = 8192
    return pl.pallas_call(
        _proj_body,
        grid=(n_rows // block,),
        in_specs=[
            pl.BlockSpec((block, SHORT), lambda i: (i, 0)),
            pl.BlockSpec((SHORT, DIM), lambda i: (0, 0)),
            pl.BlockSpec((1, DIM), lambda i: (0, 0)),
        ],
        out_specs=pl.BlockSpec((block, DIM), lambda i: (i, 0)),
        out_shape=jax.ShapeDtypeStruct((n_rows, DIM), jnp.bfloat16),
    )(x, wt, b2)


def kernel(ids, embed, W, b):
    B, L = ids.shape
    n_rows = B * L
    num_emb = embed.shape[0]
    ids_flat = ids.reshape(n_rows).astype(jnp.int32)
    t = jax.lax.bitcast_convert_type(embed.reshape(num_emb, WORDS, 2), jnp.int32)
    t128 = jax.lax.optimization_barrier(t.reshape(num_emb // 8, 8 * WORDS))
    table_i32 = t128.reshape(num_emb, WORDS)
    x_i32 = _sc_gather(ids_flat, table_i32, n_rows)
    x = jax.lax.bitcast_convert_type(x_i32, jnp.bfloat16).reshape(n_rows, SHORT)
    wt = W.astype(jnp.bfloat16).T
    b2 = b.astype(jnp.bfloat16).reshape(1, DIM)
    out = _tc_project(x, wt, b2, n_rows)
    return out.reshape(B, L, DIM)


# speedup vs baseline: 5.4513x; 1.0008x over previous
import functools

import jax
import jax.numpy as jnp
from jax import lax
from jax.experimental import pallas as pl
from jax.experimental.pallas import tpu as pltpu
from jax.experimental.pallas import tpu_sc as plsc

NUM_WORKERS = 32
SHORT = 32
DIM = 128


WORDS = SHORT // 2


def _sc_gather(ids_flat, table, n_rows):
    b_per_w = n_rows // NUM_WORKERS
    mesh = plsc.VectorSubcoreMesh(core_axis_name="c", subcore_axis_name="s")

    @functools.partial(
        pl.kernel,
        mesh=mesh,
        out_type=jax.ShapeDtypeStruct((n_rows, WORDS), jnp.int32),
        scratch_types=[
            pltpu.VMEM((b_per_w,), jnp.int32),
            pltpu.VMEM((b_per_w, WORDS), jnp.int32),
            pltpu.SemaphoreType.DMA,
        ],
        compiler_params=pltpu.CompilerParams(use_tc_tiling_on_sc=False),
    )
    def gather_kernel(ids_hbm, table_hbm, out_hbm, idx_v, rows_v, sem):
        wid = lax.axis_index("s") * 2 + lax.axis_index("c")
        base = wid * b_per_w
        pltpu.sync_copy(ids_hbm.at[pl.ds(base, b_per_w)], idx_v)
        pltpu.async_copy(table_hbm.at[idx_v], rows_v, sem).wait()
        pltpu.sync_copy(rows_v, out_hbm.at[pl.ds(base, b_per_w)])

    return gather_kernel(ids_flat, table)


def _proj_body(x_ref, w_ref, b_ref, o_ref):
    acc = jnp.dot(x_ref[...], w_ref[...], preferred_element_type=jnp.float32)
    o_ref[...] = (acc + b_ref[...]).astype(jnp.bfloat16)


def _tc_project(x, wt, b2, n_rows):
    block = 8192
    return pl.pallas_call(
        _proj_body,
        grid=(n_rows // block,),
        in_specs=[
            pl.BlockSpec((block, SHORT), lambda i: (i, 0)),
            pl.BlockSpec((SHORT, DIM), lambda i: (0, 0)),
            pl.BlockSpec((1, DIM), lambda i: (0, 0)),
        ],
        out_specs=pl.BlockSpec((block, DIM), lambda i: (i, 0)),
        out_shape=jax.ShapeDtypeStruct((n_rows, DIM), jnp.bfloat16),
    )(x, wt, b2)


def kernel(ids, embed, W, b):
    B, L = ids.shape
    n_rows = B * L
    num_emb = embed.shape[0]
    ids_flat = ids.reshape(n_rows).astype(jnp.int32)
    eT = embed.T.reshape(WORDS, 2, num_emb).transpose(0, 2, 1)
    table_i32 = jax.lax.bitcast_convert_type(eT, jnp.int32).T
    x_i32 = _sc_gather(ids_flat, table_i32, n_rows)
    x = jax.lax.bitcast_convert_type(x_i32, jnp.bfloat16).reshape(n_rows, SHORT)
    wt = W.astype(jnp.bfloat16).T
    b2 = b.astype(jnp.bfloat16).reshape(1, DIM)
    out = _tc_project(x, wt, b2, n_rows)
    return out.reshape(B, L, DIM)


# speedup vs baseline: 7.6367x; 1.4009x over previous
import functools

import jax
import jax.numpy as jnp
from jax import lax
from jax.experimental import pallas as pl
from jax.experimental.pallas import tpu as pltpu
from jax.experimental.pallas import tpu_sc as plsc

NUM_WORKERS = 32
SHORT = 32
DIM = 128


WORDS = SHORT // 2


def _sc_gather(ids_flat, table, n_rows):
    b_per_w = n_rows // NUM_WORKERS
    mesh = plsc.VectorSubcoreMesh(core_axis_name="c", subcore_axis_name="s")

    @functools.partial(
        pl.kernel,
        mesh=mesh,
        out_type=jax.ShapeDtypeStruct((n_rows, WORDS), jnp.int32),
        scratch_types=[
            pltpu.VMEM((b_per_w,), jnp.int32),
            pltpu.VMEM((b_per_w, WORDS), jnp.int32),
            pltpu.SemaphoreType.DMA,
        ],
        compiler_params=pltpu.CompilerParams(use_tc_tiling_on_sc=False),
    )
    def gather_kernel(ids_hbm, table_hbm, out_hbm, idx_v, rows_v, sem):
        wid = lax.axis_index("s") * 2 + lax.axis_index("c")
        base = wid * b_per_w
        pltpu.sync_copy(ids_hbm.at[pl.ds(base, b_per_w)], idx_v)
        pltpu.async_copy(table_hbm.at[idx_v], rows_v, sem).wait()
        pltpu.sync_copy(rows_v, out_hbm.at[pl.ds(base, b_per_w)])

    return gather_kernel(ids_flat, table)


def _repack_body(et_ref, o_ref):
    o_ref[...] = et_ref.bitcast(jnp.int32)[...]


def _tc_wordview(eT, num_emb):
    c = 65536
    return pl.pallas_call(
        _repack_body,
        grid=(num_emb // c,),
        in_specs=[pl.BlockSpec((2 * WORDS, c), lambda i: (0, i))],
        out_specs=pl.BlockSpec((WORDS, c), lambda i: (0, i)),
        out_shape=jax.ShapeDtypeStruct((WORDS, num_emb), jnp.int32),
    )(eT)


def _proj_body(x_ref, w_ref, b_ref, o_ref):
    acc = jnp.dot(x_ref[...], w_ref[...], preferred_element_type=jnp.float32)
    o_ref[...] = (acc + b_ref[...]).astype(jnp.bfloat16)


def _tc_project(x, wt, b2, n_rows):
    block = 8192
    return pl.pallas_call(
        _proj_body,
        grid=(n_rows // block,),
        in_specs=[
            pl.BlockSpec((block, SHORT), lambda i: (i, 0)),
            pl.BlockSpec((SHORT, DIM), lambda i: (0, 0)),
            pl.BlockSpec((1, DIM), lambda i: (0, 0)),
        ],
        out_specs=pl.BlockSpec((block, DIM), lambda i: (i, 0)),
        out_shape=jax.ShapeDtypeStruct((n_rows, DIM), jnp.bfloat16),
    )(x, wt, b2)


def kernel(ids, embed, W, b):
    B, L = ids.shape
    n_rows = B * L
    num_emb = embed.shape[0]
    ids_flat = ids.reshape(n_rows).astype(jnp.int32)
    t16 = _tc_wordview(embed.T, num_emb)
    table_i32 = t16.T
    x_i32 = _sc_gather(ids_flat, table_i32, n_rows)
    x = jax.lax.bitcast_convert_type(x_i32, jnp.bfloat16).reshape(n_rows, SHORT)
    wt = W.astype(jnp.bfloat16).T
    b2 = b.astype(jnp.bfloat16).reshape(1, DIM)
    out = _tc_project(x, wt, b2, n_rows)
    return out.reshape(B, L, DIM)


# speedup vs baseline: 11.1017x; 1.4537x over previous
import functools

import jax
import jax.numpy as jnp
from jax import lax
from jax.experimental import pallas as pl
from jax.experimental.pallas import tpu as pltpu
from jax.experimental.pallas import tpu_sc as plsc

NUM_WORKERS = 32
SHORT = 32
DIM = 128


WORDS = SHORT // 2


def _sc_gather(ids_flat, table, n_rows):
    b_per_w = n_rows // NUM_WORKERS
    mesh = plsc.VectorSubcoreMesh(core_axis_name="c", subcore_axis_name="s")

    @functools.partial(
        pl.kernel,
        mesh=mesh,
        out_type=jax.ShapeDtypeStruct((n_rows, WORDS), jnp.int32),
        scratch_types=[
            pltpu.VMEM((b_per_w,), jnp.int32),
            pltpu.VMEM((b_per_w, WORDS), jnp.int32),
            pltpu.SemaphoreType.DMA,
        ],
        compiler_params=pltpu.CompilerParams(use_tc_tiling_on_sc=False),
    )
    def gather_kernel(ids_hbm, table_hbm, out_hbm, idx_v, rows_v, sem):
        wid = lax.axis_index("s") * 2 + lax.axis_index("c")
        base = wid * b_per_w
        pltpu.sync_copy(ids_hbm.at[pl.ds(base, b_per_w)], idx_v)
        pltpu.async_copy(table_hbm.at[idx_v], rows_v, sem).wait()
        pltpu.sync_copy(rows_v, out_hbm.at[pl.ds(base, b_per_w)])

    return gather_kernel(ids_flat, table)


def _repack_body(et_ref, o_ref):
    o_ref[...] = et_ref.bitcast(jnp.int32)[...]


def _tc_wordview(eT, num_emb):
    c = 65536
    return pl.pallas_call(
        _repack_body,
        grid=(pl.cdiv(num_emb, c),),
        in_specs=[pl.BlockSpec((2 * WORDS, c), lambda i: (0, i))],
        out_specs=pl.BlockSpec((WORDS, c), lambda i: (0, i)),
        out_shape=jax.ShapeDtypeStruct((WORDS, num_emb), jnp.int32),
    )(eT)


def _proj_body(x_ref, we_ref, wo_ref, b_ref, o_ref):
    xw = x_ref[...]
    e = jax.lax.bitcast_convert_type(xw << 16, jnp.float32)
    o = jax.lax.bitcast_convert_type(
        xw & jnp.int32(-65536), jnp.float32
    )
    acc = jnp.dot(e, we_ref[...], preferred_element_type=jnp.float32)
    acc += jnp.dot(o, wo_ref[...], preferred_element_type=jnp.float32)
    o_ref[...] = (acc + b_ref[...]).astype(jnp.bfloat16)


def _tc_project(xw, we, wo, b8, n_lines):
    block = 3200
    return pl.pallas_call(
        _proj_body,
        grid=(n_lines // block,),
        in_specs=[
            pl.BlockSpec((block, 8 * WORDS), lambda i: (i, 0)),
            pl.BlockSpec((8 * WORDS, 8 * DIM), lambda i: (0, 0)),
            pl.BlockSpec((8 * WORDS, 8 * DIM), lambda i: (0, 0)),
            pl.BlockSpec((1, 8 * DIM), lambda i: (0, 0)),
        ],
        out_specs=pl.BlockSpec((block, 8 * DIM), lambda i: (i, 0)),
        out_shape=jax.ShapeDtypeStruct((n_lines, 8 * DIM), jnp.bfloat16),
    )(xw, we, wo, b8)


def kernel(ids, embed, W, b):
    B, L = ids.shape
    n_rows = B * L
    num_emb = embed.shape[0]
    ids_flat = ids.T.reshape(n_rows).astype(jnp.int32)
    t16 = _tc_wordview(embed.T, num_emb)
    table_i32 = t16.T
    x2 = _sc_gather(ids_flat, table_i32, n_rows)
    n_lines = n_rows // 8
    xw = x2.reshape(n_lines, 8 * WORDS)
    wc = W.astype(jnp.bfloat16).astype(jnp.float32)
    we = jnp.kron(jnp.eye(8, dtype=jnp.float32), wc[:, 0::2].T)
    wo = jnp.kron(jnp.eye(8, dtype=jnp.float32), wc[:, 1::2].T)
    b8 = jnp.tile(
        b.astype(jnp.bfloat16).astype(jnp.float32), 8
    ).reshape(1, 8 * DIM)
    out8 = _tc_project(xw, we, wo, b8, n_lines)
    out = out8.reshape(n_rows, DIM)
    return out.reshape(L, B, DIM).transpose(1, 0, 2)


# speedup vs baseline: 12.3444x; 1.1119x over previous
import functools

import jax
import jax.numpy as jnp
from jax import lax
from jax.experimental import pallas as pl
from jax.experimental.pallas import tpu as pltpu
from jax.experimental.pallas import tpu_sc as plsc

NUM_WORKERS = 32
SHORT = 32
DIM = 128


WORDS = SHORT // 2


def _sc_gather(ids_flat, table, n_rows):
    b_per_w = n_rows // NUM_WORKERS
    mesh = plsc.VectorSubcoreMesh(core_axis_name="c", subcore_axis_name="s")

    @functools.partial(
        pl.kernel,
        mesh=mesh,
        out_type=jax.ShapeDtypeStruct((n_rows, WORDS), jnp.int32),
        scratch_types=[
            pltpu.VMEM((b_per_w,), jnp.int32),
            pltpu.VMEM((b_per_w, WORDS), jnp.int32),
            pltpu.SemaphoreType.DMA,
        ],
        compiler_params=pltpu.CompilerParams(use_tc_tiling_on_sc=False),
    )
    def gather_kernel(ids_hbm, table_hbm, out_hbm, idx_v, rows_v, sem):
        wid = lax.axis_index("s") * 2 + lax.axis_index("c")
        base = wid * b_per_w
        pltpu.sync_copy(ids_hbm.at[pl.ds(base, b_per_w)], idx_v)
        pltpu.async_copy(table_hbm.at[idx_v], rows_v, sem).wait()
        pltpu.sync_copy(rows_v, out_hbm.at[pl.ds(base, b_per_w)])

    return gather_kernel(ids_flat, table)


def _repack_body(et_ref, o_ref):
    o_ref[...] = et_ref.bitcast(jnp.int32)[...].T


def _tc_wordview(eT, num_emb):
    c = 8192
    return pl.pallas_call(
        _repack_body,
        grid=(pl.cdiv(num_emb, c),),
        in_specs=[pl.BlockSpec((2 * WORDS, c), lambda i: (0, i))],
        out_specs=pl.BlockSpec((c, WORDS), lambda i: (i, 0)),
        out_shape=jax.ShapeDtypeStruct((num_emb, WORDS), jnp.int32),
    )(eT)


def _proj_body(x_ref, we_ref, wo_ref, b_ref, o_ref):
    xw = x_ref[...]
    e = jax.lax.bitcast_convert_type(xw << 16, jnp.float32)
    o = jax.lax.bitcast_convert_type(
        xw & jnp.int32(-65536), jnp.float32
    )
    acc = jnp.dot(e, we_ref[...], preferred_element_type=jnp.float32)
    acc += jnp.dot(o, wo_ref[...], preferred_element_type=jnp.float32)
    o_ref[...] = (acc + b_ref[...]).astype(jnp.bfloat16)


def _tc_project(xw, we, wo, b8, n_lines):
    block = 3200
    return pl.pallas_call(
        _proj_body,
        grid=(n_lines // block,),
        in_specs=[
            pl.BlockSpec((block, 8 * WORDS), lambda i: (i, 0)),
            pl.BlockSpec((8 * WORDS, 8 * DIM), lambda i: (0, 0)),
            pl.BlockSpec((8 * WORDS, 8 * DIM), lambda i: (0, 0)),
            pl.BlockSpec((1, 8 * DIM), lambda i: (0, 0)),
        ],
        out_specs=pl.BlockSpec((block, 8 * DIM), lambda i: (i, 0)),
        out_shape=jax.ShapeDtypeStruct((n_lines, 8 * DIM), jnp.bfloat16),
    )(xw, we, wo, b8)


def kernel(ids, embed, W, b):
    B, L = ids.shape
    n_rows = B * L
    num_emb = embed.shape[0]
    ids_flat = ids.T.reshape(n_rows).astype(jnp.int32)
    table_i32 = _tc_wordview(embed.T, num_emb)
    x2 = _sc_gather(ids_flat, table_i32, n_rows)
    n_lines = n_rows // 8
    xw = x2.reshape(n_lines, 8 * WORDS)
    wc = W.astype(jnp.bfloat16).astype(jnp.float32)
    we = jnp.kron(jnp.eye(8, dtype=jnp.float32), wc[:, 0::2].T)
    wo = jnp.kron(jnp.eye(8, dtype=jnp.float32), wc[:, 1::2].T)
    b8 = jnp.tile(
        b.astype(jnp.bfloat16).astype(jnp.float32), 8
    ).reshape(1, 8 * DIM)
    out8 = _tc_project(xw, we, wo, b8, n_lines)
    out = out8.reshape(n_rows, DIM)
    return out.reshape(L, B, DIM).transpose(1, 0, 2)


# speedup vs baseline: 13.0308x; 1.0556x over previous
import functools

import jax
import jax.numpy as jnp
from jax import lax
from jax.experimental import pallas as pl
from jax.experimental.pallas import tpu as pltpu
from jax.experimental.pallas import tpu_sc as plsc

NUM_WORKERS = 32
SHORT = 32
DIM = 128


WORDS = SHORT // 2


def _sc_gather(ids_flat, table, n_rows):
    b_per_w = n_rows // NUM_WORKERS
    mesh = plsc.VectorSubcoreMesh(core_axis_name="c", subcore_axis_name="s")

    @functools.partial(
        pl.kernel,
        mesh=mesh,
        out_type=jax.ShapeDtypeStruct((n_rows, WORDS), jnp.int32),
        scratch_types=[
            pltpu.VMEM((b_per_w,), jnp.int32),
            pltpu.VMEM((b_per_w, WORDS), jnp.int32),
            pltpu.SemaphoreType.DMA,
        ],
        compiler_params=pltpu.CompilerParams(use_tc_tiling_on_sc=False),
    )
    def gather_kernel(ids_hbm, table_hbm, out_hbm, idx_v, rows_v, sem):
        wid = lax.axis_index("s") * 2 + lax.axis_index("c")
        base = wid * b_per_w
        pltpu.sync_copy(ids_hbm.at[pl.ds(base, b_per_w)], idx_v)
        pltpu.async_copy(table_hbm.at[idx_v], rows_v, sem).wait()
        pltpu.sync_copy(rows_v, out_hbm.at[pl.ds(base, b_per_w)])

    return gather_kernel(ids_flat, table)


def _repack_body(et_ref, o_ref):
    o_ref[...] = et_ref.bitcast(jnp.int32)[...].T


def _tc_wordview(eT, num_emb):
    c = 16384
    return pl.pallas_call(
        _repack_body,
        grid=(pl.cdiv(num_emb, c),),
        in_specs=[pl.BlockSpec((2 * WORDS, c), lambda i: (0, i))],
        out_specs=pl.BlockSpec((c, WORDS), lambda i: (i, 0)),
        out_shape=jax.ShapeDtypeStruct((num_emb, WORDS), jnp.int32),
    )(eT)


def _proj_body(x_ref, we_ref, wo_ref, b_ref, o_ref):
    xw = x_ref[...]
    e = jax.lax.bitcast_convert_type(xw << 16, jnp.float32)
    o = jax.lax.bitcast_convert_type(
        xw & jnp.int32(-65536), jnp.float32
    )
    acc = jnp.dot(e, we_ref[...], preferred_element_type=jnp.float32)
    acc += jnp.dot(o, wo_ref[...], preferred_element_type=jnp.float32)
    o_ref[...] = (acc + b_ref[...]).astype(jnp.bfloat16)


def _tc_project(xw, we, wo, b8, n_lines):
    block = 3200
    return pl.pallas_call(
        _proj_body,
        grid=(n_lines // block,),
        in_specs=[
            pl.BlockSpec((block, 8 * WORDS), lambda i: (i, 0)),
            pl.BlockSpec((8 * WORDS, 8 * DIM), lambda i: (0, 0)),
            pl.BlockSpec((8 * WORDS, 8 * DIM), lambda i: (0, 0)),
            pl.BlockSpec((1, 8 * DIM), lambda i: (0, 0)),
        ],
        out_specs=pl.BlockSpec((block, 8 * DIM), lambda i: (i, 0)),
        out_shape=jax.ShapeDtypeStruct((n_lines, 8 * DIM), jnp.bfloat16),
    )(xw, we, wo, b8)


def kernel(ids, embed, W, b):
    B, L = ids.shape
    n_rows = B * L
    num_emb = embed.shape[0]
    ids_flat = ids.T.reshape(n_rows).astype(jnp.int32)
    table_i32 = _tc_wordview(embed.T, num_emb)
    x2 = _sc_gather(ids_flat, table_i32, n_rows)
    n_lines = n_rows // 8
    xw = x2.reshape(n_lines, 8 * WORDS)
    wc = W.astype(jnp.bfloat16).astype(jnp.float32)
    we = jnp.kron(jnp.eye(8, dtype=jnp.float32), wc[:, 0::2].T)
    wo = jnp.kron(jnp.eye(8, dtype=jnp.float32), wc[:, 1::2].T)
    b8 = jnp.tile(
        b.astype(jnp.bfloat16).astype(jnp.float32), 8
    ).reshape(1, 8 * DIM)
    out8 = _tc_project(xw, we, wo, b8, n_lines)
    out = out8.reshape(n_rows, DIM)
    return out.reshape(L, B, DIM).transpose(1, 0, 2)
